# baseline (speedup 1.0000x reference)
import functools

import jax
import jax.numpy as jnp
from jax import lax
from jax.experimental import pallas as pl
from jax.experimental.pallas import tpu as pltpu
from jax.experimental.pallas import tpu_sc as plsc

N_NODES = 10000
N_EDGES = 160000
D_NODE = 128
D_EDGE = 16
MSG_D = 32
LAYERS = 3
MID = 128
TGT = 1
N_GRAPHS = 16

CHUNK = 128
N_CHUNKS = N_EDGES // CHUNK
SC_CORES = 2
SC_SUBCORES = 16
N_TILES = SC_CORES * SC_SUBCORES
ZROWS = N_NODES // SC_SUBCORES


def _lin_relu_body(x_ref, w_ref, b_ref, o_ref):
    o_ref[...] = jax.nn.relu(
        jnp.dot(x_ref[...], w_ref[...], preferred_element_type=jnp.float32)
        + b_ref[...])


def _lin_relu(x, w, b2d, block_rows):
    rows, k = x.shape
    n = w.shape[1]
    grid = rows // block_rows
    return pl.pallas_call(
        _lin_relu_body,
        grid=(grid,),
        in_specs=[
            pl.BlockSpec((block_rows, k), lambda i: (i, 0)),
            pl.BlockSpec((k, n), lambda i: (0, 0)),
            pl.BlockSpec((1, n), lambda i: (0, 0)),
        ],
        out_specs=pl.BlockSpec((block_rows, n), lambda i: (i, 0)),
        out_shape=jax.ShapeDtypeStruct((rows, n), jnp.float32),
    )(x, w, b2d)


def _contract(ew, hs):
    prod = ew * jnp.repeat(hs, MSG_D, axis=1)
    s = prod[:, 0:128]
    for i4 in range(1, MSG_D * MSG_D // 128):
        s = s + prod[:, 128 * i4:128 * (i4 + 1)]
    return ((s[:, 0:MSG_D] + s[:, MSG_D:2 * MSG_D])
            + (s[:, 2 * MSG_D:3 * MSG_D] + s[:, 3 * MSG_D:4 * MSG_D]))


def _msg_first_body(t_ref, hs_ref, we2_ref, be2_ref, o_ref, ew_ref):
    ew = jnp.dot(t_ref[...], we2_ref[...],
                 preferred_element_type=jnp.float32) + be2_ref[...]
    ew_ref[...] = ew
    o_ref[...] = _contract(ew, hs_ref[...])


def _msg_first(t, hs, w_e2, b_e2_2d):
    block = 2000
    grid = N_EDGES // block
    return pl.pallas_call(
        _msg_first_body,
        grid=(grid,),
        in_specs=[
            pl.BlockSpec((block, MSG_D), lambda i: (i, 0)),
            pl.BlockSpec((block, MSG_D), lambda i: (i, 0)),
            pl.BlockSpec((MSG_D, MSG_D * MSG_D), lambda i: (0, 0)),
            pl.BlockSpec((1, MSG_D * MSG_D), lambda i: (0, 0)),
        ],
        out_specs=[
            pl.BlockSpec((block, MSG_D), lambda i: (i, 0)),
            pl.BlockSpec((block, MSG_D * MSG_D), lambda i: (i, 0)),
        ],
        out_shape=[
            jax.ShapeDtypeStruct((N_EDGES, MSG_D), jnp.float32),
            jax.ShapeDtypeStruct((N_EDGES, MSG_D * MSG_D), jnp.float32),
        ],
        compiler_params=pltpu.CompilerParams(vmem_limit_bytes=100 * 2**20),
    )(t, hs, w_e2, b_e2_2d)


def _msg_cached_body(ew_ref, hs_ref, o_ref):
    o_ref[...] = _contract(ew_ref[...], hs_ref[...])


def _msg_cached(ew, hs):
    block = 2000
    grid = N_EDGES // block
    return pl.pallas_call(
        _msg_cached_body,
        grid=(grid,),
        in_specs=[
            pl.BlockSpec((block, MSG_D * MSG_D), lambda i: (i, 0)),
            pl.BlockSpec((block, MSG_D), lambda i: (i, 0)),
        ],
        out_specs=pl.BlockSpec((block, MSG_D), lambda i: (i, 0)),
        out_shape=jax.ShapeDtypeStruct((N_EDGES, MSG_D), jnp.float32),
        compiler_params=pltpu.CompilerParams(vmem_limit_bytes=100 * 2**20),
    )(ew, hs)


def _gru_body(a0_ref, a1_ref, h_ref, bconv_ref, wih_ref, bih_ref, whh_ref,
              bhh_ref, o_ref):
    m = jax.nn.relu(a0_ref[...] + a1_ref[...] + bconv_ref[...])
    h = h_ref[...]
    gi = jnp.dot(m, wih_ref[...], preferred_element_type=jnp.float32) + bih_ref[...]
    gh = jnp.dot(h, whh_ref[...], preferred_element_type=jnp.float32) + bhh_ref[...]
    d = MSG_D
    r = jax.nn.sigmoid(gi[:, 0:d] + gh[:, 0:d])
    z = jax.nn.sigmoid(gi[:, d:2 * d] + gh[:, d:2 * d])
    n = jnp.tanh(gi[:, 2 * d:3 * d] + r * gh[:, 2 * d:3 * d])
    o_ref[...] = (1.0 - z) * n + z * h


def _gru_update(a0, a1, h, bconv2d, w_ih, b_ih2d, w_hh, b_hh2d):
    d = MSG_D
    return pl.pallas_call(
        _gru_body,
        grid=(1,),
        in_specs=[
            pl.BlockSpec((N_NODES, d), lambda i: (0, 0)),
            pl.BlockSpec((N_NODES, d), lambda i: (0, 0)),
            pl.BlockSpec((N_NODES, d), lambda i: (0, 0)),
            pl.BlockSpec((1, d), lambda i: (0, 0)),
            pl.BlockSpec((d, 3 * d), lambda i: (0, 0)),
            pl.BlockSpec((1, 3 * d), lambda i: (0, 0)),
            pl.BlockSpec((d, 3 * d), lambda i: (0, 0)),
            pl.BlockSpec((1, 3 * d), lambda i: (0, 0)),
        ],
        out_specs=pl.BlockSpec((N_NODES, d), lambda i: (0, 0)),
        out_shape=jax.ShapeDtypeStruct((N_NODES, d), jnp.float32),
    )(a0, a1, h, bconv2d, w_ih, b_ih2d, w_hh, b_hh2d)


def _readout_body(h_ref, gid_ref, wawt_ref, baw_ref, wt1_ref, bt1_ref,
                  wt2_ref, bt2_ref, o_ref):
    h = h_ref[...]
    w = jax.nn.sigmoid(
        jnp.dot(h, wawt_ref[...], preferred_element_type=jnp.float32)
        + baw_ref[...])
    hw = h * w
    gids = gid_ref[...]
    neg = jnp.float32(-1e30)
    sums = []
    maxes = []
    for g in range(N_GRAPHS):
        mask = gids == g
        sums.append(jnp.sum(jnp.where(mask, hw, 0.0), axis=0, keepdims=True))
        maxes.append(jnp.max(jnp.where(mask, h, neg), axis=0, keepdims=True))
    h_sum = jnp.concatenate(sums, axis=0)
    h_max = jnp.concatenate(maxes, axis=0)
    emb = jnp.concatenate([h_sum, h_max], axis=1)
    y = jnp.dot(emb, wt1_ref[...], preferred_element_type=jnp.float32) + bt1_ref[...]
    y = jnp.dot(y, wt2_ref[...], preferred_element_type=jnp.float32) + bt2_ref[...]
    o_ref[...] = jax.nn.sigmoid(y)


def _readout(h, gid2d, w_aw_t, b_aw2d, w_t1, b_t12d, w_t2, b_t22d):
    d = MSG_D
    return pl.pallas_call(
        _readout_body,
        grid=(1,),
        in_specs=[
            pl.BlockSpec((N_NODES, d), lambda i: (0, 0)),
            pl.BlockSpec((N_NODES, 1), lambda i: (0, 0)),
            pl.BlockSpec((d, 1), lambda i: (0, 0)),
            pl.BlockSpec((1, 1), lambda i: (0, 0)),
            pl.BlockSpec((2 * d, MID), lambda i: (0, 0)),
            pl.BlockSpec((1, MID), lambda i: (0, 0)),
            pl.BlockSpec((MID, TGT), lambda i: (0, 0)),
            pl.BlockSpec((1, TGT), lambda i: (0, 0)),
        ],
        out_specs=pl.BlockSpec((N_GRAPHS, TGT), lambda i: (0, 0)),
        out_shape=jax.ShapeDtypeStruct((N_GRAPHS, TGT), jnp.float32),
    )(h, gid2d, w_aw_t, b_aw2d, w_t1, b_t12d, w_t2, b_t22d)


def _tile_chunk_range(wid):
    base = N_CHUNKS // N_TILES
    rem = N_CHUNKS % N_TILES
    start = wid * base + jnp.minimum(wid, rem)
    cnt = base + jnp.where(wid < rem, 1, 0)
    return start, cnt


def _sc_gather_body(h_hbm, src2d_hbm, out_hbm, idx_v, rows_v, sem):
    wid = lax.axis_index("s") * SC_CORES + lax.axis_index("c")
    start, cnt = _tile_chunk_range(wid)

    def body(j, carry):
        c = start + j
        pltpu.sync_copy(src2d_hbm.at[c], idx_v)
        pltpu.async_copy(h_hbm.at[idx_v], rows_v, sem).wait()
        pltpu.sync_copy(rows_v, out_hbm.at[pl.ds(c * CHUNK, CHUNK)])
        return carry

    lax.fori_loop(0, cnt, body, 0)


@functools.lru_cache(maxsize=None)
def _sc_gather_kernel():
    return pl.kernel(
        _sc_gather_body,
        out_type=jax.ShapeDtypeStruct((N_EDGES, MSG_D), jnp.float32),
        mesh=plsc.VectorSubcoreMesh(core_axis_name="c", subcore_axis_name="s",
                                    num_cores=SC_CORES,
                                    num_subcores=SC_SUBCORES),
        scratch_types=[
            pltpu.VMEM((CHUNK,), jnp.int32),
            pltpu.VMEM((CHUNK, MSG_D), jnp.float32),
            pltpu.SemaphoreType.DMA,
        ],
        compiler_params=pltpu.CompilerParams(use_tc_tiling_on_sc=False),
    )


def _sc_gather(h, src2d):
    return _sc_gather_kernel()(h, src2d)


def _sc_scatter_body(msg_hbm, dst2d_hbm, zeros_hbm, out_hbm, idxrow_v, rows_v,
                     agg_sh):
    cid = lax.axis_index("c")
    sid = lax.axis_index("s")
    wid = sid * SC_CORES + cid
    pltpu.sync_copy(zeros_hbm.at[pl.ds(sid * ZROWS, ZROWS)],
                    agg_sh.at[pl.ds(sid * ZROWS, ZROWS)])
    plsc.subcore_barrier()
    start, cnt = _tile_chunk_range(wid)

    def body(j, carry):
        c = start + j
        pltpu.sync_copy(dst2d_hbm.at[pl.ds(c, 1)], idxrow_v)
        pltpu.sync_copy(msg_hbm.at[pl.ds(c * CHUNK, CHUNK)], rows_v)
        pltpu.sync_copy(rows_v, agg_sh.at[idxrow_v.at[0]], add=True)
        return carry

    lax.fori_loop(0, cnt, body, 0)
    plsc.subcore_barrier()
    pltpu.sync_copy(agg_sh.at[pl.ds(sid * ZROWS, ZROWS)],
                    out_hbm.at[cid].at[pl.ds(sid * ZROWS, ZROWS)])


@functools.lru_cache(maxsize=None)
def _sc_scatter_kernel():
    return pl.kernel(
        _sc_scatter_body,
        out_type=jax.ShapeDtypeStruct((SC_CORES, N_NODES, MSG_D),
                                      jnp.float32),
        mesh=plsc.VectorSubcoreMesh(core_axis_name="c", subcore_axis_name="s",
                                    num_cores=SC_CORES,
                                    num_subcores=SC_SUBCORES),
        scratch_types=[
            pltpu.VMEM((1, CHUNK), jnp.int32),
            pltpu.VMEM((CHUNK, MSG_D), jnp.float32),
            pltpu.VMEM_SHARED((N_NODES, MSG_D), jnp.float32),
        ],
        compiler_params=pltpu.CompilerParams(use_tc_tiling_on_sc=False),
    )


def _sc_scatter(msg, dst2d, zeros_n):
    return _sc_scatter_kernel()(msg, dst2d, zeros_n)


def kernel(x, edge_attr, edge_index, node_graph_ids, W_proj, b_proj, W_e1,
           b_e1, W_e2, b_e2, b_conv, W_ih, b_ih, W_hh, b_hh, W_aw, b_aw,
           W_t1, b_t1, W_t2, b_t2):
    src2d = edge_index[0].reshape(N_CHUNKS, CHUNK)
    dst2d = edge_index[1].reshape(N_CHUNKS, CHUNK)
    gid2d = node_graph_ids.reshape(N_NODES, 1)
    zeros_n = jnp.zeros((N_NODES, MSG_D), jnp.float32)

    h = _lin_relu(x, W_proj, b_proj.reshape(1, -1), block_rows=2000)
    t = _lin_relu(edge_attr, W_e1, b_e1.reshape(1, -1), block_rows=4000)

    be2_2d = b_e2.reshape(1, -1)
    bconv2d = b_conv.reshape(1, -1)
    bih2d = b_ih.reshape(1, -1)
    bhh2d = b_hh.reshape(1, -1)
    ew = None
    for layer in range(LAYERS):
        hs = _sc_gather(h, src2d)
        if layer == 0:
            msg, ew = _msg_first(t, hs, W_e2, be2_2d)
        else:
            msg = _msg_cached(ew, hs)
        agg2 = _sc_scatter(msg, dst2d, zeros_n)
        h = _gru_update(agg2[0], agg2[1], h, bconv2d, W_ih, bih2d, W_hh,
                        bhh2d)

    return _readout(h, gid2d, W_aw, b_aw.reshape(1, -1),
                    W_t1, b_t1.reshape(1, -1), W_t2, b_t2.reshape(1, -1))


# speedup vs baseline: 1.0351x; 1.0351x over previous
import functools

import jax
import jax.numpy as jnp
from jax import lax
from jax.experimental import pallas as pl
from jax.experimental.pallas import tpu as pltpu
from jax.experimental.pallas import tpu_sc as plsc

N_NODES = 10000
N_EDGES = 160000
D_NODE = 128
D_EDGE = 16
MSG_D = 32
LAYERS = 3
MID = 128
TGT = 1
N_GRAPHS = 16

CHUNK = 128
N_CHUNKS = N_EDGES // CHUNK
SC_CORES = 2
SC_SUBCORES = 16
N_TILES = SC_CORES * SC_SUBCORES
ZROWS = N_NODES // SC_SUBCORES


def _lin_relu_body(x_ref, w_ref, b_ref, o_ref):
    o_ref[...] = jax.nn.relu(
        jnp.dot(x_ref[...], w_ref[...], preferred_element_type=jnp.float32)
        + b_ref[...])


def _lin_relu(x, w, b2d, block_rows):
    rows, k = x.shape
    n = w.shape[1]
    grid = rows // block_rows
    return pl.pallas_call(
        _lin_relu_body,
        grid=(grid,),
        in_specs=[
            pl.BlockSpec((block_rows, k), lambda i: (i, 0)),
            pl.BlockSpec((k, n), lambda i: (0, 0)),
            pl.BlockSpec((1, n), lambda i: (0, 0)),
        ],
        out_specs=pl.BlockSpec((block_rows, n), lambda i: (i, 0)),
        out_shape=jax.ShapeDtypeStruct((rows, n), jnp.float32),
    )(x, w, b2d)


def _contract(ew, hs):
    prod = ew * jnp.repeat(hs, MSG_D, axis=1)
    s = prod[:, 0:128]
    for i4 in range(1, MSG_D * MSG_D // 128):
        s = s + prod[:, 128 * i4:128 * (i4 + 1)]
    return ((s[:, 0:MSG_D] + s[:, MSG_D:2 * MSG_D])
            + (s[:, 2 * MSG_D:3 * MSG_D] + s[:, 3 * MSG_D:4 * MSG_D]))


def _msg_first_body(t_ref, hs_ref, we2_ref, be2_ref, o_ref, ew_ref):
    ew = jnp.dot(t_ref[...], we2_ref[...],
                 preferred_element_type=jnp.float32) + be2_ref[...]
    ew_ref[...] = ew
    o_ref[...] = _contract(ew, hs_ref[...])


def _msg_first(t, hs, w_e2, b_e2_2d):
    block = 2000
    grid = N_EDGES // block
    return pl.pallas_call(
        _msg_first_body,
        grid=(grid,),
        in_specs=[
            pl.BlockSpec((block, MSG_D), lambda i: (i, 0)),
            pl.BlockSpec((block, MSG_D), lambda i: (i, 0)),
            pl.BlockSpec((MSG_D, MSG_D * MSG_D), lambda i: (0, 0)),
            pl.BlockSpec((1, MSG_D * MSG_D), lambda i: (0, 0)),
        ],
        out_specs=[
            pl.BlockSpec((block, MSG_D), lambda i: (i, 0)),
            pl.BlockSpec((block, MSG_D * MSG_D), lambda i: (i, 0)),
        ],
        out_shape=[
            jax.ShapeDtypeStruct((N_EDGES, MSG_D), jnp.float32),
            jax.ShapeDtypeStruct((N_EDGES, MSG_D * MSG_D), jnp.float32),
        ],
        compiler_params=pltpu.CompilerParams(vmem_limit_bytes=100 * 2**20),
    )(t, hs, w_e2, b_e2_2d)


def _msg_cached_body(ew_ref, hs_ref, o_ref):
    o_ref[...] = _contract(ew_ref[...], hs_ref[...])


def _msg_cached(ew, hs):
    block = 2000
    grid = N_EDGES // block
    return pl.pallas_call(
        _msg_cached_body,
        grid=(grid,),
        in_specs=[
            pl.BlockSpec((block, MSG_D * MSG_D), lambda i: (i, 0)),
            pl.BlockSpec((block, MSG_D), lambda i: (i, 0)),
        ],
        out_specs=pl.BlockSpec((block, MSG_D), lambda i: (i, 0)),
        out_shape=jax.ShapeDtypeStruct((N_EDGES, MSG_D), jnp.float32),
        compiler_params=pltpu.CompilerParams(vmem_limit_bytes=100 * 2**20),
    )(ew, hs)


def _gru_body(a0_ref, a1_ref, h_ref, bconv_ref, wih_ref, bih_ref, whh_ref,
              bhh_ref, o_ref):
    m = jax.nn.relu(a0_ref[...] + a1_ref[...] + bconv_ref[...])
    h = h_ref[...]
    gi = jnp.dot(m, wih_ref[...], preferred_element_type=jnp.float32) + bih_ref[...]
    gh = jnp.dot(h, whh_ref[...], preferred_element_type=jnp.float32) + bhh_ref[...]
    d = MSG_D
    r = jax.nn.sigmoid(gi[:, 0:d] + gh[:, 0:d])
    z = jax.nn.sigmoid(gi[:, d:2 * d] + gh[:, d:2 * d])
    n = jnp.tanh(gi[:, 2 * d:3 * d] + r * gh[:, 2 * d:3 * d])
    o_ref[...] = (1.0 - z) * n + z * h


def _gru_update(a0, a1, h, bconv2d, w_ih, b_ih2d, w_hh, b_hh2d):
    d = MSG_D
    return pl.pallas_call(
        _gru_body,
        grid=(1,),
        in_specs=[
            pl.BlockSpec((N_NODES, d), lambda i: (0, 0)),
            pl.BlockSpec((N_NODES, d), lambda i: (0, 0)),
            pl.BlockSpec((N_NODES, d), lambda i: (0, 0)),
            pl.BlockSpec((1, d), lambda i: (0, 0)),
            pl.BlockSpec((d, 3 * d), lambda i: (0, 0)),
            pl.BlockSpec((1, 3 * d), lambda i: (0, 0)),
            pl.BlockSpec((d, 3 * d), lambda i: (0, 0)),
            pl.BlockSpec((1, 3 * d), lambda i: (0, 0)),
        ],
        out_specs=pl.BlockSpec((N_NODES, d), lambda i: (0, 0)),
        out_shape=jax.ShapeDtypeStruct((N_NODES, d), jnp.float32),
    )(a0, a1, h, bconv2d, w_ih, b_ih2d, w_hh, b_hh2d)


def _readout_body(h_ref, gid_ref, wawt_ref, baw_ref, wt1_ref, bt1_ref,
                  wt2_ref, bt2_ref, o_ref):
    h = h_ref[...]
    w = jax.nn.sigmoid(
        jnp.dot(h, wawt_ref[...], preferred_element_type=jnp.float32)
        + baw_ref[...])
    hw = h * w
    gids = gid_ref[...]
    neg = jnp.float32(-1e30)
    sums = []
    maxes = []
    for g in range(N_GRAPHS):
        mask = gids == g
        sums.append(jnp.sum(jnp.where(mask, hw, 0.0), axis=0, keepdims=True))
        maxes.append(jnp.max(jnp.where(mask, h, neg), axis=0, keepdims=True))
    h_sum = jnp.concatenate(sums, axis=0)
    h_max = jnp.concatenate(maxes, axis=0)
    emb = jnp.concatenate([h_sum, h_max], axis=1)
    y = jnp.dot(emb, wt1_ref[...], preferred_element_type=jnp.float32) + bt1_ref[...]
    y = jnp.dot(y, wt2_ref[...], preferred_element_type=jnp.float32) + bt2_ref[...]
    o_ref[...] = jax.nn.sigmoid(y)


def _readout(h, gid2d, w_aw_t, b_aw2d, w_t1, b_t12d, w_t2, b_t22d):
    d = MSG_D
    return pl.pallas_call(
        _readout_body,
        grid=(1,),
        in_specs=[
            pl.BlockSpec((N_NODES, d), lambda i: (0, 0)),
            pl.BlockSpec((N_NODES, 1), lambda i: (0, 0)),
            pl.BlockSpec((d, 1), lambda i: (0, 0)),
            pl.BlockSpec((1, 1), lambda i: (0, 0)),
            pl.BlockSpec((2 * d, MID), lambda i: (0, 0)),
            pl.BlockSpec((1, MID), lambda i: (0, 0)),
            pl.BlockSpec((MID, TGT), lambda i: (0, 0)),
            pl.BlockSpec((1, TGT), lambda i: (0, 0)),
        ],
        out_specs=pl.BlockSpec((N_GRAPHS, TGT), lambda i: (0, 0)),
        out_shape=jax.ShapeDtypeStruct((N_GRAPHS, TGT), jnp.float32),
    )(h, gid2d, w_aw_t, b_aw2d, w_t1, b_t12d, w_t2, b_t22d)


def _tile_chunk_range(wid):
    base = N_CHUNKS // N_TILES
    rem = N_CHUNKS % N_TILES
    start = wid * base + jnp.minimum(wid, rem)
    cnt = base + jnp.where(wid < rem, 1, 0)
    return start, cnt


_CPT = N_CHUNKS // N_TILES
_CREM = N_CHUNKS % N_TILES


def _sc_gather_body(h_hbm, src2d_hbm, out_hbm, idx2d_v, rows0_v, rows1_v,
                    xidx_v, sem0, sem1):
    wid = lax.axis_index("s") * SC_CORES + lax.axis_index("c")
    start = wid * _CPT
    pltpu.sync_copy(src2d_hbm.at[pl.ds(start, _CPT)], idx2d_v)

    @pl.when(wid < _CREM)
    def _():
        c = N_TILES * _CPT + wid
        pltpu.sync_copy(src2d_hbm.at[pl.ds(c, 1)], xidx_v)
        pltpu.async_copy(h_hbm.at[xidx_v.at[0]], rows0_v, sem0).wait()
        pltpu.sync_copy(rows0_v, out_hbm.at[pl.ds(c * CHUNK, CHUNK)])

    bufs = (rows0_v, rows1_v)
    sems = (sem0, sem1)
    descs = [None, None]
    for g in range(_CPT):
        descs[g % 2] = pltpu.async_copy(h_hbm.at[idx2d_v.at[g]], bufs[g % 2],
                                        sems[g % 2])
        if g >= 1:
            descs[(g - 1) % 2].wait()
            pltpu.sync_copy(bufs[(g - 1) % 2],
                            out_hbm.at[pl.ds((start + g - 1) * CHUNK, CHUNK)])
    descs[(_CPT - 1) % 2].wait()
    pltpu.sync_copy(bufs[(_CPT - 1) % 2],
                    out_hbm.at[pl.ds((start + _CPT - 1) * CHUNK, CHUNK)])


@functools.lru_cache(maxsize=None)
def _sc_gather_kernel():
    return pl.kernel(
        _sc_gather_body,
        out_type=jax.ShapeDtypeStruct((N_EDGES, MSG_D), jnp.float32),
        mesh=plsc.VectorSubcoreMesh(core_axis_name="c", subcore_axis_name="s",
                                    num_cores=SC_CORES,
                                    num_subcores=SC_SUBCORES),
        scratch_types=[
            pltpu.VMEM((_CPT, CHUNK), jnp.int32),
            pltpu.VMEM((CHUNK, MSG_D), jnp.float32),
            pltpu.VMEM((CHUNK, MSG_D), jnp.float32),
            pltpu.VMEM((1, CHUNK), jnp.int32),
            pltpu.SemaphoreType.DMA,
            pltpu.SemaphoreType.DMA,
        ],
        compiler_params=pltpu.CompilerParams(use_tc_tiling_on_sc=False),
    )


def _sc_gather(h, src2d):
    return _sc_gather_kernel()(h, src2d)


def _sc_scatter_body(msg_hbm, dst2d_hbm, zeros_hbm, out_hbm, idx2d_v, rows0_v,
                     rows1_v, xidx_v, sem0, sem1, agg_sh):
    cid = lax.axis_index("c")
    sid = lax.axis_index("s")
    wid = sid * SC_CORES + cid
    start = wid * _CPT
    pltpu.sync_copy(zeros_hbm.at[pl.ds(sid * ZROWS, ZROWS)],
                    agg_sh.at[pl.ds(sid * ZROWS, ZROWS)])
    pltpu.sync_copy(dst2d_hbm.at[pl.ds(start, _CPT)], idx2d_v)
    plsc.subcore_barrier()

    @pl.when(wid < _CREM)
    def _():
        c = N_TILES * _CPT + wid
        pltpu.sync_copy(dst2d_hbm.at[pl.ds(c, 1)], xidx_v)
        pltpu.sync_copy(msg_hbm.at[pl.ds(c * CHUNK, CHUNK)], rows0_v)
        pltpu.sync_copy(rows0_v, agg_sh.at[xidx_v.at[0]], add=True)

    bufs = (rows0_v, rows1_v)
    sems = (sem0, sem1)
    descs = [None, None]
    for g in range(_CPT):
        descs[g % 2] = pltpu.async_copy(
            msg_hbm.at[pl.ds((start + g) * CHUNK, CHUNK)], bufs[g % 2],
            sems[g % 2])
        if g >= 1:
            descs[(g - 1) % 2].wait()
            pltpu.sync_copy(bufs[(g - 1) % 2],
                            agg_sh.at[idx2d_v.at[g - 1]], add=True)
    descs[(_CPT - 1) % 2].wait()
    pltpu.sync_copy(bufs[(_CPT - 1) % 2],
                    agg_sh.at[idx2d_v.at[_CPT - 1]], add=True)

    plsc.subcore_barrier()
    pltpu.sync_copy(agg_sh.at[pl.ds(sid * ZROWS, ZROWS)],
                    out_hbm.at[cid].at[pl.ds(sid * ZROWS, ZROWS)])


@functools.lru_cache(maxsize=None)
def _sc_scatter_kernel():
    return pl.kernel(
        _sc_scatter_body,
        out_type=jax.ShapeDtypeStruct((SC_CORES, N_NODES, MSG_D),
                                      jnp.float32),
        mesh=plsc.VectorSubcoreMesh(core_axis_name="c", subcore_axis_name="s",
                                    num_cores=SC_CORES,
                                    num_subcores=SC_SUBCORES),
        scratch_types=[
            pltpu.VMEM((_CPT, CHUNK), jnp.int32),
            pltpu.VMEM((CHUNK, MSG_D), jnp.float32),
            pltpu.VMEM((CHUNK, MSG_D), jnp.float32),
            pltpu.VMEM((1, CHUNK), jnp.int32),
            pltpu.SemaphoreType.DMA,
            pltpu.SemaphoreType.DMA,
            pltpu.VMEM_SHARED((N_NODES, MSG_D), jnp.float32),
        ],
        compiler_params=pltpu.CompilerParams(use_tc_tiling_on_sc=False),
    )


def _sc_scatter(msg, dst2d, zeros_n):
    return _sc_scatter_kernel()(msg, dst2d, zeros_n)


def kernel(x, edge_attr, edge_index, node_graph_ids, W_proj, b_proj, W_e1,
           b_e1, W_e2, b_e2, b_conv, W_ih, b_ih, W_hh, b_hh, W_aw, b_aw,
           W_t1, b_t1, W_t2, b_t2):
    src2d = edge_index[0].reshape(N_CHUNKS, CHUNK)
    dst2d = edge_index[1].reshape(N_CHUNKS, CHUNK)
    gid2d = node_graph_ids.reshape(N_NODES, 1)
    zeros_n = jnp.zeros((N_NODES, MSG_D), jnp.float32)

    h = _lin_relu(x, W_proj, b_proj.reshape(1, -1), block_rows=2000)
    t = _lin_relu(edge_attr, W_e1, b_e1.reshape(1, -1), block_rows=4000)

    be2_2d = b_e2.reshape(1, -1)
    bconv2d = b_conv.reshape(1, -1)
    bih2d = b_ih.reshape(1, -1)
    bhh2d = b_hh.reshape(1, -1)
    ew = None
    for layer in range(LAYERS):
        hs = _sc_gather(h, src2d)
        if layer == 0:
            msg, ew = _msg_first(t, hs, W_e2, be2_2d)
        else:
            msg = _msg_cached(ew, hs)
        agg2 = _sc_scatter(msg, dst2d, zeros_n)
        h = _gru_update(agg2[0], agg2[1], h, bconv2d, W_ih, bih2d, W_hh,
                        bhh2d)

    return _readout(h, gid2d, W_aw, b_aw.reshape(1, -1),
                    W_t1, b_t1.reshape(1, -1), W_t2, b_t2.reshape(1, -1))


# speedup vs baseline: 1.0663x; 1.0301x over previous
import functools

import jax
import jax.numpy as jnp
from jax import lax
from jax.experimental import pallas as pl
from jax.experimental.pallas import tpu as pltpu
from jax.experimental.pallas import tpu_sc as plsc

N_NODES = 10000
N_EDGES = 160000
D_NODE = 128
D_EDGE = 16
MSG_D = 32
LAYERS = 3
MID = 128
TGT = 1
N_GRAPHS = 16

CHUNK = 128
N_CHUNKS = N_EDGES // CHUNK
SC_CORES = 2
SC_SUBCORES = 16
N_TILES = SC_CORES * SC_SUBCORES
ZROWS = N_NODES // SC_SUBCORES


def _lin_relu_body(x_ref, w_ref, b_ref, o_ref):
    o_ref[...] = jax.nn.relu(
        jnp.dot(x_ref[...], w_ref[...], preferred_element_type=jnp.float32)
        + b_ref[...])


def _lin_relu(x, w, b2d, block_rows):
    rows, k = x.shape
    n = w.shape[1]
    grid = rows // block_rows
    return pl.pallas_call(
        _lin_relu_body,
        grid=(grid,),
        in_specs=[
            pl.BlockSpec((block_rows, k), lambda i: (i, 0)),
            pl.BlockSpec((k, n), lambda i: (0, 0)),
            pl.BlockSpec((1, n), lambda i: (0, 0)),
        ],
        out_specs=pl.BlockSpec((block_rows, n), lambda i: (i, 0)),
        out_shape=jax.ShapeDtypeStruct((rows, n), jnp.float32),
    )(x, w, b2d)


def _contract(ewp, hs):
    prod = ewp * jnp.tile(hs, (1, MSG_D))
    return prod.reshape(-1, MSG_D, MSG_D).sum(axis=2)


def _msg_first_body(t_ref, hs_ref, we2_ref, be2_ref, o_ref, ew_ref):
    ew = jnp.dot(t_ref[...], we2_ref[...],
                 preferred_element_type=jnp.float32) + be2_ref[...]
    ew_ref[...] = ew
    o_ref[...] = _contract(ew, hs_ref[...])


def _msg_first(t, hs, w_e2, b_e2_2d):
    block = 2000
    grid = N_EDGES // block
    return pl.pallas_call(
        _msg_first_body,
        grid=(grid,),
        in_specs=[
            pl.BlockSpec((block, MSG_D), lambda i: (i, 0)),
            pl.BlockSpec((block, MSG_D), lambda i: (i, 0)),
            pl.BlockSpec((MSG_D, MSG_D * MSG_D), lambda i: (0, 0)),
            pl.BlockSpec((1, MSG_D * MSG_D), lambda i: (0, 0)),
        ],
        out_specs=[
            pl.BlockSpec((block, MSG_D), lambda i: (i, 0)),
            pl.BlockSpec((block, MSG_D * MSG_D), lambda i: (i, 0)),
        ],
        out_shape=[
            jax.ShapeDtypeStruct((N_EDGES, MSG_D), jnp.float32),
            jax.ShapeDtypeStruct((N_EDGES, MSG_D * MSG_D), jnp.float32),
        ],
        compiler_params=pltpu.CompilerParams(vmem_limit_bytes=100 * 2**20),
    )(t, hs, w_e2, b_e2_2d)


def _msg_cached_body(ew_ref, hs_ref, o_ref):
    o_ref[...] = _contract(ew_ref[...], hs_ref[...])


def _msg_cached(ew, hs):
    block = 2000
    grid = N_EDGES // block
    return pl.pallas_call(
        _msg_cached_body,
        grid=(grid,),
        in_specs=[
            pl.BlockSpec((block, MSG_D * MSG_D), lambda i: (i, 0)),
            pl.BlockSpec((block, MSG_D), lambda i: (i, 0)),
        ],
        out_specs=pl.BlockSpec((block, MSG_D), lambda i: (i, 0)),
        out_shape=jax.ShapeDtypeStruct((N_EDGES, MSG_D), jnp.float32),
        compiler_params=pltpu.CompilerParams(vmem_limit_bytes=100 * 2**20),
    )(ew, hs)


def _gru_body(a0_ref, a1_ref, h_ref, bconv_ref, wih_ref, bih_ref, whh_ref,
              bhh_ref, o_ref):
    m = jax.nn.relu(a0_ref[...] + a1_ref[...] + bconv_ref[...])
    h = h_ref[...]
    gi = jnp.dot(m, wih_ref[...], preferred_element_type=jnp.float32) + bih_ref[...]
    gh = jnp.dot(h, whh_ref[...], preferred_element_type=jnp.float32) + bhh_ref[...]
    d = MSG_D
    r = jax.nn.sigmoid(gi[:, 0:d] + gh[:, 0:d])
    z = jax.nn.sigmoid(gi[:, d:2 * d] + gh[:, d:2 * d])
    n = jnp.tanh(gi[:, 2 * d:3 * d] + r * gh[:, 2 * d:3 * d])
    o_ref[...] = (1.0 - z) * n + z * h


def _gru_update(a0, a1, h, bconv2d, w_ih, b_ih2d, w_hh, b_hh2d):
    d = MSG_D
    return pl.pallas_call(
        _gru_body,
        grid=(1,),
        in_specs=[
            pl.BlockSpec((N_NODES, d), lambda i: (0, 0)),
            pl.BlockSpec((N_NODES, d), lambda i: (0, 0)),
            pl.BlockSpec((N_NODES, d), lambda i: (0, 0)),
            pl.BlockSpec((1, d), lambda i: (0, 0)),
            pl.BlockSpec((d, 3 * d), lambda i: (0, 0)),
            pl.BlockSpec((1, 3 * d), lambda i: (0, 0)),
            pl.BlockSpec((d, 3 * d), lambda i: (0, 0)),
            pl.BlockSpec((1, 3 * d), lambda i: (0, 0)),
        ],
        out_specs=pl.BlockSpec((N_NODES, d), lambda i: (0, 0)),
        out_shape=jax.ShapeDtypeStruct((N_NODES, d), jnp.float32),
    )(a0, a1, h, bconv2d, w_ih, b_ih2d, w_hh, b_hh2d)


def _readout_body(h_ref, gid_ref, wawt_ref, baw_ref, wt1_ref, bt1_ref,
                  wt2_ref, bt2_ref, o_ref):
    h = h_ref[...]
    w = jax.nn.sigmoid(
        jnp.dot(h, wawt_ref[...], preferred_element_type=jnp.float32)
        + baw_ref[...])
    hw = h * w
    gids = gid_ref[...]
    neg = jnp.float32(-1e30)
    sums = []
    maxes = []
    for g in range(N_GRAPHS):
        mask = gids == g
        sums.append(jnp.sum(jnp.where(mask, hw, 0.0), axis=0, keepdims=True))
        maxes.append(jnp.max(jnp.where(mask, h, neg), axis=0, keepdims=True))
    h_sum = jnp.concatenate(sums, axis=0)
    h_max = jnp.concatenate(maxes, axis=0)
    emb = jnp.concatenate([h_sum, h_max], axis=1)
    y = jnp.dot(emb, wt1_ref[...], preferred_element_type=jnp.float32) + bt1_ref[...]
    y = jnp.dot(y, wt2_ref[...], preferred_element_type=jnp.float32) + bt2_ref[...]
    o_ref[...] = jax.nn.sigmoid(y)


def _readout(h, gid2d, w_aw_t, b_aw2d, w_t1, b_t12d, w_t2, b_t22d):
    d = MSG_D
    return pl.pallas_call(
        _readout_body,
        grid=(1,),
        in_specs=[
            pl.BlockSpec((N_NODES, d), lambda i: (0, 0)),
            pl.BlockSpec((N_NODES, 1), lambda i: (0, 0)),
            pl.BlockSpec((d, 1), lambda i: (0, 0)),
            pl.BlockSpec((1, 1), lambda i: (0, 0)),
            pl.BlockSpec((2 * d, MID), lambda i: (0, 0)),
            pl.BlockSpec((1, MID), lambda i: (0, 0)),
            pl.BlockSpec((MID, TGT), lambda i: (0, 0)),
            pl.BlockSpec((1, TGT), lambda i: (0, 0)),
        ],
        out_specs=pl.BlockSpec((N_GRAPHS, TGT), lambda i: (0, 0)),
        out_shape=jax.ShapeDtypeStruct((N_GRAPHS, TGT), jnp.float32),
    )(h, gid2d, w_aw_t, b_aw2d, w_t1, b_t12d, w_t2, b_t22d)


def _tile_chunk_range(wid):
    base = N_CHUNKS // N_TILES
    rem = N_CHUNKS % N_TILES
    start = wid * base + jnp.minimum(wid, rem)
    cnt = base + jnp.where(wid < rem, 1, 0)
    return start, cnt


_CPT = N_CHUNKS // N_TILES
_CREM = N_CHUNKS % N_TILES


def _sc_gather_body(h_hbm, src2d_hbm, out_hbm, idx2d_v, rows0_v, rows1_v,
                    xidx_v, sem0, sem1):
    wid = lax.axis_index("s") * SC_CORES + lax.axis_index("c")
    start = wid * _CPT
    pltpu.sync_copy(src2d_hbm.at[pl.ds(start, _CPT)], idx2d_v)

    @pl.when(wid < _CREM)
    def _():
        c = N_TILES * _CPT + wid
        pltpu.sync_copy(src2d_hbm.at[pl.ds(c, 1)], xidx_v)
        pltpu.async_copy(h_hbm.at[xidx_v.at[0]], rows0_v, sem0).wait()
        pltpu.sync_copy(rows0_v, out_hbm.at[pl.ds(c * CHUNK, CHUNK)])

    bufs = (rows0_v, rows1_v)
    sems = (sem0, sem1)
    descs = [None, None]
    for g in range(_CPT):
        descs[g % 2] = pltpu.async_copy(h_hbm.at[idx2d_v.at[g]], bufs[g % 2],
                                        sems[g % 2])
        if g >= 1:
            descs[(g - 1) % 2].wait()
            pltpu.sync_copy(bufs[(g - 1) % 2],
                            out_hbm.at[pl.ds((start + g - 1) * CHUNK, CHUNK)])
    descs[(_CPT - 1) % 2].wait()
    pltpu.sync_copy(bufs[(_CPT - 1) % 2],
                    out_hbm.at[pl.ds((start + _CPT - 1) * CHUNK, CHUNK)])


@functools.lru_cache(maxsize=None)
def _sc_gather_kernel():
    return pl.kernel(
        _sc_gather_body,
        out_type=jax.ShapeDtypeStruct((N_EDGES, MSG_D), jnp.float32),
        mesh=plsc.VectorSubcoreMesh(core_axis_name="c", subcore_axis_name="s",
                                    num_cores=SC_CORES,
                                    num_subcores=SC_SUBCORES),
        scratch_types=[
            pltpu.VMEM((_CPT, CHUNK), jnp.int32),
            pltpu.VMEM((CHUNK, MSG_D), jnp.float32),
            pltpu.VMEM((CHUNK, MSG_D), jnp.float32),
            pltpu.VMEM((1, CHUNK), jnp.int32),
            pltpu.SemaphoreType.DMA,
            pltpu.SemaphoreType.DMA,
        ],
        compiler_params=pltpu.CompilerParams(use_tc_tiling_on_sc=False),
    )


def _sc_gather(h, src2d):
    return _sc_gather_kernel()(h, src2d)


def _sc_scatter_body(msg_hbm, dst2d_hbm, zeros_hbm, out_hbm, idx2d_v, rows0_v,
                     rows1_v, xidx_v, sem0, sem1, agg_sh):
    cid = lax.axis_index("c")
    sid = lax.axis_index("s")
    wid = sid * SC_CORES + cid
    start = wid * _CPT
    pltpu.sync_copy(zeros_hbm.at[pl.ds(sid * ZROWS, ZROWS)],
                    agg_sh.at[pl.ds(sid * ZROWS, ZROWS)])
    pltpu.sync_copy(dst2d_hbm.at[pl.ds(start, _CPT)], idx2d_v)
    plsc.subcore_barrier()

    @pl.when(wid < _CREM)
    def _():
        c = N_TILES * _CPT + wid
        pltpu.sync_copy(dst2d_hbm.at[pl.ds(c, 1)], xidx_v)
        pltpu.sync_copy(msg_hbm.at[pl.ds(c * CHUNK, CHUNK)], rows0_v)
        pltpu.sync_copy(rows0_v, agg_sh.at[xidx_v.at[0]], add=True)

    bufs = (rows0_v, rows1_v)
    sems = (sem0, sem1)
    descs = [None, None]
    for g in range(_CPT):
        descs[g % 2] = pltpu.async_copy(
            msg_hbm.at[pl.ds((start + g) * CHUNK, CHUNK)], bufs[g % 2],
            sems[g % 2])
        if g >= 1:
            descs[(g - 1) % 2].wait()
            pltpu.sync_copy(bufs[(g - 1) % 2],
                            agg_sh.at[idx2d_v.at[g - 1]], add=True)
    descs[(_CPT - 1) % 2].wait()
    pltpu.sync_copy(bufs[(_CPT - 1) % 2],
                    agg_sh.at[idx2d_v.at[_CPT - 1]], add=True)

    plsc.subcore_barrier()
    pltpu.sync_copy(agg_sh.at[pl.ds(sid * ZROWS, ZROWS)],
                    out_hbm.at[cid].at[pl.ds(sid * ZROWS, ZROWS)])


@functools.lru_cache(maxsize=None)
def _sc_scatter_kernel():
    return pl.kernel(
        _sc_scatter_body,
        out_type=jax.ShapeDtypeStruct((SC_CORES, N_NODES, MSG_D),
                                      jnp.float32),
        mesh=plsc.VectorSubcoreMesh(core_axis_name="c", subcore_axis_name="s",
                                    num_cores=SC_CORES,
                                    num_subcores=SC_SUBCORES),
        scratch_types=[
            pltpu.VMEM((_CPT, CHUNK), jnp.int32),
            pltpu.VMEM((CHUNK, MSG_D), jnp.float32),
            pltpu.VMEM((CHUNK, MSG_D), jnp.float32),
            pltpu.VMEM((1, CHUNK), jnp.int32),
            pltpu.SemaphoreType.DMA,
            pltpu.SemaphoreType.DMA,
            pltpu.VMEM_SHARED((N_NODES, MSG_D), jnp.float32),
        ],
        compiler_params=pltpu.CompilerParams(use_tc_tiling_on_sc=False),
    )


def _sc_scatter(msg, dst2d, zeros_n):
    return _sc_scatter_kernel()(msg, dst2d, zeros_n)


def kernel(x, edge_attr, edge_index, node_graph_ids, W_proj, b_proj, W_e1,
           b_e1, W_e2, b_e2, b_conv, W_ih, b_ih, W_hh, b_hh, W_aw, b_aw,
           W_t1, b_t1, W_t2, b_t2):
    src2d = edge_index[0].reshape(N_CHUNKS, CHUNK)
    dst2d = edge_index[1].reshape(N_CHUNKS, CHUNK)
    gid2d = node_graph_ids.reshape(N_NODES, 1)
    zeros_n = jnp.zeros((N_NODES, MSG_D), jnp.float32)

    W_e2 = W_e2.reshape(MSG_D, MSG_D, MSG_D).transpose(0, 2, 1).reshape(
        MSG_D, MSG_D * MSG_D)
    b_e2 = b_e2.reshape(MSG_D, MSG_D).T.reshape(MSG_D * MSG_D)

    h = _lin_relu(x, W_proj, b_proj.reshape(1, -1), block_rows=2000)
    t = _lin_relu(edge_attr, W_e1, b_e1.reshape(1, -1), block_rows=4000)

    be2_2d = b_e2.reshape(1, -1)
    bconv2d = b_conv.reshape(1, -1)
    bih2d = b_ih.reshape(1, -1)
    bhh2d = b_hh.reshape(1, -1)
    ew = None
    for layer in range(LAYERS):
        hs = _sc_gather(h, src2d)
        if layer == 0:
            msg, ew = _msg_first(t, hs, W_e2, be2_2d)
        else:
            msg = _msg_cached(ew, hs)
        agg2 = _sc_scatter(msg, dst2d, zeros_n)
        h = _gru_update(agg2[0], agg2[1], h, bconv2d, W_ih, bih2d, W_hh,
                        bhh2d)

    return _readout(h, gid2d, W_aw, b_aw.reshape(1, -1),
                    W_t1, b_t1.reshape(1, -1), W_t2, b_t2.reshape(1, -1))
